# baseline (speedup 1.0000x reference)
import functools

import jax
import jax.numpy as jnp
from jax import lax
from jax.experimental import pallas as pl
from jax.experimental.pallas import tpu as pltpu
from jax.experimental.pallas import tpu_sc as plsc

NF = 10
NV = 500
H = 128
HW = H // 2
NC = 2
NS = 16
NW = NC * NS
C = 128
NSUB = C // 128


def _sc_lookup_sum(n_pad, nchunk):
    mesh = plsc.VectorSubcoreMesh(core_axis_name="c", subcore_axis_name="s")

    @functools.partial(
        pl.kernel,
        out_type=jax.ShapeDtypeStruct((n_pad, H), jnp.float32),
        mesh=mesh,
        scratch_types=[
            pltpu.VMEM((NF, NSUB, 128), jnp.int32),
            pltpu.VMEM((C, H), jnp.float32),
            pltpu.VMEM((2, C, HW), jnp.int32),
            pltpu.VMEM_SHARED((NF * NV, HW), jnp.int32),
            pltpu.SemaphoreType.DMA,
            pltpu.SemaphoreType.DMA,
        ],
        compiler_params=pltpu.CompilerParams(use_tc_tiling_on_sc=False),
    )
    def body(x_hbm, tab_hbm, out_hbm, idx_v, acc_v, gath_v, sh_tab,
             sem_0, sem_1):
        sems = [sem_0, sem_1]
        sid = lax.axis_index("s")
        wid = sid * NC + lax.axis_index("c")

        @pl.when(sid == 0)
        def _stage():
            pltpu.sync_copy(tab_hbm, sh_tab)
        plsc.subcore_barrier()

        def fire(f, k):
            cps = []
            for s in range(NSUB):
                cps.append(pltpu.async_copy(
                    sh_tab.at[idx_v.at[f, s]],
                    gath_v.at[k, pl.ds(s * 128, 128)],
                    sems[k]))
            return cps

        def combine(k, first):
            def row_body(r, carry):
                for u in range(2):
                    row = 2 * r + u
                    for v in range(HW // 16):
                        w = gath_v[k, row, pl.ds(16 * v, 16)]
                        lo = lax.bitcast_convert_type(
                            lax.shift_left(w, 16), jnp.float32)
                        hi = lax.bitcast_convert_type(
                            lax.bitwise_and(w, jnp.int32(-65536)), jnp.float32)
                        if first:
                            acc_v[row, pl.ds(32 * v, 16)] = lo
                            acc_v[row, pl.ds(32 * v + 16, 16)] = hi
                        else:
                            plsc.addupdate(acc_v.at[row, pl.ds(32 * v, 16)], lo)
                            plsc.addupdate(
                                acc_v.at[row, pl.ds(32 * v + 16, 16)], hi)
                return carry
            lax.fori_loop(0, C // 2, row_body, 0)

        def chunk_body(j, carry):
            pltpu.sync_copy(x_hbm.at[wid, j], idx_v)
            prev_cps, prev_buf = fire(0, 0), 0
            for f in range(1, NF):
                cur_buf = f % 2
                cur_cps = fire(f, cur_buf)
                for cp in prev_cps:
                    cp.wait()
                combine(prev_buf, first=(f == 1))
                prev_cps, prev_buf = cur_cps, cur_buf
            for cp in prev_cps:
                cp.wait()
            combine(prev_buf, first=False)
            pltpu.sync_copy(acc_v,
                            out_hbm.at[pl.ds(wid * (nchunk * C) + j * C, C)])
            return carry

        lax.fori_loop(0, nchunk, chunk_body, 0)

    return body


def _pack_tables(tabs):
    tab = jnp.concatenate(tabs, axis=0)
    p = jnp.arange(H)
    g, r = p // 32, p % 32
    src = g * 32 + r // 2 + (r % 2) * 16
    tab = tab[:, src].astype(jnp.bfloat16)
    tab = tab.reshape(NF * NV, HW, 2)
    return lax.bitcast_convert_type(tab, jnp.int32)


def kernel(x, emb_0, emb_1, emb_2, emb_3, emb_4, emb_5, emb_6, emb_7,
           emb_8, emb_9):
    n = x.shape[0]
    rows_per_w = -(-n // (NW * C)) * C
    nchunk = rows_per_w // C
    n_pad = NW * rows_per_w

    xi = x.reshape(n, NF)
    xi = jnp.pad(xi, ((0, n_pad - n), (0, 0)))
    xb = xi.reshape(NW, nchunk, C, NF).transpose(0, 1, 3, 2)
    xb = xb + (jnp.arange(NF, dtype=jnp.int32) * NV).reshape(1, 1, NF, 1)
    xb = xb.reshape(NW, nchunk, NF, NSUB, 128)

    tab = _pack_tables([emb_0, emb_1, emb_2, emb_3, emb_4, emb_5, emb_6,
                        emb_7, emb_8, emb_9])

    out = _sc_lookup_sum(n_pad, nchunk)(xb, tab)
    return out[:n].reshape(n, 1, H)


# speedup vs baseline: 1.7333x; 1.7333x over previous
import functools

import jax
import jax.numpy as jnp
from jax import lax
from jax.experimental import pallas as pl
from jax.experimental.pallas import tpu as pltpu
from jax.experimental.pallas import tpu_sc as plsc

NF = 10
NV = 500
H = 128
HW = H // 2
NC = 2
NS = 16
NW = NC * NS
C = 128
NSUB = C // 128


def _sc_lookup_sum(n_pad, nchunk):
    mesh = plsc.VectorSubcoreMesh(core_axis_name="c", subcore_axis_name="s")

    @functools.partial(
        pl.kernel,
        out_type=jax.ShapeDtypeStruct((n_pad, H), jnp.float32),
        mesh=mesh,
        scratch_types=[
            pltpu.VMEM((NF, NSUB, 128), jnp.int32),
            pltpu.VMEM((C, H), jnp.float32),
            pltpu.VMEM((2, C, H), jnp.float32),
            pltpu.VMEM_SHARED((NF * NV, H), jnp.float32),
            pltpu.SemaphoreType.DMA,
            pltpu.SemaphoreType.DMA,
        ],
        compiler_params=pltpu.CompilerParams(use_tc_tiling_on_sc=False),
    )
    def body(x_hbm, tab_hbm, out_hbm, idx_v, acc_v, gath_v, sh_tab,
             sem_0, sem_1):
        sems = [sem_0, sem_1]
        sid = lax.axis_index("s")
        wid = sid * NC + lax.axis_index("c")

        @pl.when(sid == 0)
        def _stage():
            pltpu.sync_copy(tab_hbm, sh_tab)
        plsc.subcore_barrier()

        def fire(f, k):
            cps = []
            for s in range(NSUB):
                cps.append(pltpu.async_copy(
                    sh_tab.at[idx_v.at[f, s]],
                    gath_v.at[k, pl.ds(s * 128, 128)],
                    sems[k]))
            return cps

        def combine(k, first):
            def row_body(r, carry):
                for u in range(2):
                    row = 2 * r + u
                    for v in range(H // 16):
                        w = gath_v[k, row, pl.ds(16 * v, 16)]
                        if first:
                            acc_v[row, pl.ds(16 * v, 16)] = w
                        else:
                            plsc.addupdate(acc_v.at[row, pl.ds(16 * v, 16)], w)
                return carry
            lax.fori_loop(0, C // 2, row_body, 0)

        def chunk_body(j, carry):
            pltpu.sync_copy(x_hbm.at[wid, j], idx_v)
            prev_cps, prev_buf = fire(0, 0), 0
            for f in range(1, NF):
                cur_buf = f % 2
                cur_cps = fire(f, cur_buf)
                for cp in prev_cps:
                    cp.wait()
                combine(prev_buf, first=(f == 1))
                prev_cps, prev_buf = cur_cps, cur_buf
            for cp in prev_cps:
                cp.wait()
            combine(prev_buf, first=False)
            pltpu.sync_copy(acc_v,
                            out_hbm.at[pl.ds(wid * (nchunk * C) + j * C, C)])
            return carry

        lax.fori_loop(0, nchunk, chunk_body, 0)

    return body


def _pack_tables(tabs):
    tab = jnp.concatenate(tabs, axis=0)
    p = jnp.arange(H)
    g, r = p // 32, p % 32
    src = g * 32 + r // 2 + (r % 2) * 16
    tab = tab[:, src].astype(jnp.bfloat16)
    tab = tab.reshape(NF * NV, HW, 2)
    return lax.bitcast_convert_type(tab, jnp.int32)


def kernel(x, emb_0, emb_1, emb_2, emb_3, emb_4, emb_5, emb_6, emb_7,
           emb_8, emb_9):
    n = x.shape[0]
    rows_per_w = -(-n // (NW * C)) * C
    nchunk = rows_per_w // C
    n_pad = NW * rows_per_w

    xi = x.reshape(n, NF)
    xi = jnp.pad(xi, ((0, n_pad - n), (0, 0)))
    xb = xi.reshape(NW, nchunk, C, NF).transpose(0, 1, 3, 2)
    xb = xb + (jnp.arange(NF, dtype=jnp.int32) * NV).reshape(1, 1, NF, 1)
    xb = xb.reshape(NW, nchunk, NF, NSUB, 128)

    tab = jnp.concatenate([emb_0, emb_1, emb_2, emb_3, emb_4, emb_5, emb_6,
                           emb_7, emb_8, emb_9], axis=0)

    out = _sc_lookup_sum(n_pad, nchunk)(xb, tab)
    return out[:n].reshape(n, 1, H)


# speedup vs baseline: 2.4753x; 1.4281x over previous
import functools

import jax
import jax.numpy as jnp
from jax import lax
from jax.experimental import pallas as pl
from jax.experimental.pallas import tpu as pltpu
from jax.experimental.pallas import tpu_sc as plsc

NF = 10
NV = 500
H = 128
HW = H // 2
NC = 2
NS = 16
NW = NC * NS
C = 128
NSUB = C // 128


def _sc_lookup_sum(n_pad, nchunk):
    mesh = plsc.VectorSubcoreMesh(core_axis_name="c", subcore_axis_name="s")

    @functools.partial(
        pl.kernel,
        out_type=jax.ShapeDtypeStruct((n_pad, H), jnp.float32),
        mesh=mesh,
        scratch_types=[
            pltpu.VMEM((NF, NSUB, 128), jnp.int32),
            pltpu.VMEM((C, H), jnp.float32),
            pltpu.VMEM((2, C, H), jnp.float32),
            pltpu.VMEM_SHARED((NF * NV, H), jnp.float32),
            pltpu.SemaphoreType.DMA,
            pltpu.SemaphoreType.DMA,
        ],
        compiler_params=pltpu.CompilerParams(use_tc_tiling_on_sc=False),
    )
    def body(x_hbm, tab_hbm, out_hbm, idx_v, acc_v, gath_v, sh_tab,
             sem_0, sem_1):
        sems = [sem_0, sem_1]
        sid = lax.axis_index("s")
        wid = sid * NC + lax.axis_index("c")

        @pl.when(sid == 0)
        def _stage():
            pltpu.sync_copy(tab_hbm, sh_tab)
        plsc.subcore_barrier()

        def fire(f, k):
            cps = []
            for s in range(NSUB):
                cps.append(pltpu.async_copy(
                    sh_tab.at[idx_v.at[f, s]],
                    gath_v.at[k, pl.ds(s * 128, 128)],
                    sems[k]))
            return cps

        def fire_add(f):
            cps = []
            for s in range(NSUB):
                cps.append(pltpu.async_copy(
                    sh_tab.at[idx_v.at[f, s]],
                    acc_v.at[pl.ds(s * 128, 128)],
                    sem_1, add=True))
            return cps

        def combine(k, first):
            def row_body(r, carry):
                for u in range(2):
                    row = 2 * r + u
                    for v in range(H // 16):
                        w = gath_v[k, row, pl.ds(16 * v, 16)]
                        if first:
                            acc_v[row, pl.ds(16 * v, 16)] = w
                        else:
                            plsc.addupdate(acc_v.at[row, pl.ds(16 * v, 16)], w)
                return carry
            lax.fori_loop(0, C // 2, row_body, 0)

        def chunk_body(j, carry):
            pltpu.sync_copy(x_hbm.at[wid, j], idx_v)
            for cp in fire(0, 0):
                cp.wait()
            combine(0, first=True)
            pend = []
            for f in range(1, NF):
                pend += fire_add(f)
            for cp in pend:
                cp.wait()
            pltpu.sync_copy(acc_v,
                            out_hbm.at[pl.ds(wid * (nchunk * C) + j * C, C)])
            return carry

        lax.fori_loop(0, nchunk, chunk_body, 0)

    return body


def _pack_tables(tabs):
    tab = jnp.concatenate(tabs, axis=0)
    p = jnp.arange(H)
    g, r = p // 32, p % 32
    src = g * 32 + r // 2 + (r % 2) * 16
    tab = tab[:, src].astype(jnp.bfloat16)
    tab = tab.reshape(NF * NV, HW, 2)
    return lax.bitcast_convert_type(tab, jnp.int32)


def kernel(x, emb_0, emb_1, emb_2, emb_3, emb_4, emb_5, emb_6, emb_7,
           emb_8, emb_9):
    n = x.shape[0]
    rows_per_w = -(-n // (NW * C)) * C
    nchunk = rows_per_w // C
    n_pad = NW * rows_per_w

    xi = x.reshape(n, NF)
    xi = jnp.pad(xi, ((0, n_pad - n), (0, 0)))
    xb = xi.reshape(NW, nchunk, C, NF).transpose(0, 1, 3, 2)
    xb = xb + (jnp.arange(NF, dtype=jnp.int32) * NV).reshape(1, 1, NF, 1)
    xb = xb.reshape(NW, nchunk, NF, NSUB, 128)

    tab = jnp.concatenate([emb_0, emb_1, emb_2, emb_3, emb_4, emb_5, emb_6,
                           emb_7, emb_8, emb_9], axis=0)

    out = _sc_lookup_sum(n_pad, nchunk)(xb, tab)
    return out[:n].reshape(n, 1, H)


# speedup vs baseline: 2.8671x; 1.1583x over previous
import functools

import jax
import jax.numpy as jnp
from jax import lax
from jax.experimental import pallas as pl
from jax.experimental.pallas import tpu as pltpu
from jax.experimental.pallas import tpu_sc as plsc

NF = 10
NV = 500
H = 128
NC = 2
NS = 16
NW = NC * NS
C = 128


def _sc_lookup_sum(n_pad, nchunk):
    mesh = plsc.VectorSubcoreMesh(core_axis_name="c", subcore_axis_name="s")

    @functools.partial(
        pl.kernel,
        out_type=jax.ShapeDtypeStruct((n_pad, H), jnp.float32),
        mesh=mesh,
        scratch_types=[
            pltpu.VMEM((2, NF, 1, C), jnp.int32),
            pltpu.VMEM((2, C, H), jnp.float32),
            pltpu.VMEM_SHARED((NF * NV, H), jnp.float32),
            pltpu.SemaphoreType.DMA,
            pltpu.SemaphoreType.DMA,
            pltpu.SemaphoreType.DMA,
        ],
    )
    def body(x_hbm, tab_hbm, out_hbm, idx_v, acc_v, sh_tab,
             sem_g, sem_i, sem_o):
        sid = lax.axis_index("s")
        wid = sid * NC + lax.axis_index("c")

        @pl.when(sid == 0)
        def _stage():
            pltpu.sync_copy(tab_hbm, sh_tab)
        plsc.subcore_barrier()

        pltpu.async_copy(x_hbm.at[wid, 0], idx_v.at[0], sem_i)

        def chunk_body(j, carry):
            p = lax.rem(j, 2)
            idx = idx_v.at[p]
            acc = acc_v.at[p]

            pltpu.make_async_copy(x_hbm.at[wid, j], idx, sem_i).wait()

            @pl.when(j + 1 < nchunk)
            def _prefetch():
                pltpu.async_copy(x_hbm.at[wid, j + 1], idx_v.at[1 - p], sem_i)

            @pl.when(j >= 2)
            def _drain():
                pltpu.make_async_copy(
                    acc, out_hbm.at[pl.ds(0, C)], sem_o).wait()

            pltpu.async_copy(sh_tab.at[idx.at[0, 0]], acc, sem_g).wait()
            cps = [pltpu.async_copy(sh_tab.at[idx.at[f, 0]], acc, sem_g,
                                    add=True)
                   for f in range(1, NF)]
            for cp in cps:
                cp.wait()

            pltpu.async_copy(
                acc, out_hbm.at[pl.ds(wid * (nchunk * C) + j * C, C)], sem_o)
            return carry

        lax.fori_loop(0, nchunk, chunk_body, 0)

        for _ in range(2):
            pltpu.make_async_copy(
                acc_v.at[0], out_hbm.at[pl.ds(0, C)], sem_o).wait()

    return body


def kernel(x, emb_0, emb_1, emb_2, emb_3, emb_4, emb_5, emb_6, emb_7,
           emb_8, emb_9):
    n = x.shape[0]
    rows_per_w = -(-n // (NW * C)) * C
    nchunk = rows_per_w // C
    n_pad = NW * rows_per_w

    xi = x.reshape(n, NF)
    xi = jnp.pad(xi, ((0, n_pad - n), (0, 0)))
    xb = xi.reshape(NW, nchunk, C, NF).transpose(0, 1, 3, 2)
    xb = xb + (jnp.arange(NF, dtype=jnp.int32) * NV).reshape(1, 1, NF, 1)
    xb = xb.reshape(NW, nchunk, NF, 1, C)

    tab = jnp.concatenate([emb_0, emb_1, emb_2, emb_3, emb_4, emb_5, emb_6,
                           emb_7, emb_8, emb_9], axis=0)

    out = _sc_lookup_sum(n_pad, nchunk)(xb, tab)
    return out[:n].reshape(n, 1, H)


# speedup vs baseline: 3.3307x; 1.1617x over previous
import functools

import jax
import jax.numpy as jnp
from jax import lax
from jax.experimental import pallas as pl
from jax.experimental.pallas import tpu as pltpu
from jax.experimental.pallas import tpu_sc as plsc

NF = 10
NV = 500
H = 128
NC = 2
NS = 16
NW = NC * NS
C = 128


def _sc_lookup_sum(n, nch, slots):
    mesh = plsc.VectorSubcoreMesh(core_axis_name="c", subcore_axis_name="s")
    rem_workers = nch - NW * (slots - 1)

    @functools.partial(
        pl.kernel,
        out_type=jax.ShapeDtypeStruct((n, H), jnp.float32),
        mesh=mesh,
        scratch_types=[
            pltpu.VMEM((2, NF, 1, C), jnp.int32),
            pltpu.VMEM((2, C, H), jnp.float32),
            pltpu.VMEM_SHARED((NF * NV, H), jnp.float32),
            pltpu.SemaphoreType.DMA,
            pltpu.SemaphoreType.DMA,
            pltpu.SemaphoreType.DMA,
        ],
    )
    def body(x_hbm, tab_hbm, out_hbm, idx_v, acc_v, sh_tab,
             sem_g, sem_i, sem_o):
        sid = lax.axis_index("s")
        wid = sid * NC + lax.axis_index("c")

        @pl.when(sid == 0)
        def _stage():
            pltpu.sync_copy(tab_hbm, sh_tab)
        plsc.subcore_barrier()

        pltpu.async_copy(x_hbm.at[0, wid], idx_v.at[0], sem_i)

        def chunk_body(j, carry):
            p = lax.rem(j, 2)
            idx = idx_v.at[p]
            acc = acc_v.at[p]
            g = j * NW + wid
            base = jnp.minimum(g * C, n - C)

            pltpu.make_async_copy(x_hbm.at[j, wid], idx, sem_i).wait()

            @pl.when(j + 1 < slots)
            def _prefetch():
                pltpu.async_copy(x_hbm.at[j + 1, wid], idx_v.at[1 - p], sem_i)

            @pl.when(j >= 2)
            def _drain():
                pltpu.make_async_copy(
                    acc, out_hbm.at[pl.ds(0, C)], sem_o).wait()

            @pl.when(g < nch)
            def _work():
                pltpu.async_copy(sh_tab.at[idx.at[0, 0]], acc, sem_g).wait()
                cps = [pltpu.async_copy(sh_tab.at[idx.at[f, 0]], acc, sem_g,
                                        add=True)
                       for f in range(1, NF)]
                for cp in cps:
                    cp.wait()
                pltpu.async_copy(acc, out_hbm.at[pl.ds(base, C)], sem_o)
            return carry

        lax.fori_loop(0, slots, chunk_body, 0)

        pltpu.make_async_copy(acc_v.at[0], out_hbm.at[pl.ds(0, C)],
                              sem_o).wait()

        @pl.when(wid < rem_workers)
        def _last_drain():
            pltpu.make_async_copy(acc_v.at[0], out_hbm.at[pl.ds(0, C)],
                                  sem_o).wait()

    return body


def kernel(x, emb_0, emb_1, emb_2, emb_3, emb_4, emb_5, emb_6, emb_7,
           emb_8, emb_9):
    n = x.shape[0]
    full, rem = divmod(n, C)
    nch = full + (1 if rem else 0)
    slots = -(-nch // NW)
    xi = x.reshape(n, NF)
    if rem:
        xi = jnp.concatenate([xi[:full * C], xi[n - C:]], axis=0)
    if slots * NW > nch:
        xi = jnp.concatenate(
            [xi, jnp.zeros(((slots * NW - nch) * C, NF), jnp.int32)], axis=0)
    xb = xi.reshape(slots, NW, C, NF).transpose(0, 1, 3, 2)
    xb = xb + (jnp.arange(NF, dtype=jnp.int32) * NV).reshape(1, 1, NF, 1)
    xb = xb.reshape(slots, NW, NF, 1, C)

    tab = jnp.concatenate([emb_0, emb_1, emb_2, emb_3, emb_4, emb_5, emb_6,
                           emb_7, emb_8, emb_9], axis=0)

    out = _sc_lookup_sum(n, nch, slots)(xb, tab)
    return out.reshape(n, 1, H)
